# baseline (speedup 1.0000x reference)
import functools

import jax
import jax.numpy as jnp
from jax import lax
from jax.experimental import pallas as pl
from jax.experimental.pallas import tpu as pltpu
from jax.experimental.pallas import tpu_sc as plsc

NUM_EXPERTS = 64
D_MODEL = 2048
BLOCK_T = 1024

N_TOK = 16384
NC, NS, L = 2, 16, 16
NW = NC * NS
TPW = N_TOK // NW
CHUNK = 128
N_CHUNKS = TPW // CHUNK


def _matmul_body(x_ref, w_ref, lg_ref):
    logits = jax.lax.dot_general(
        w_ref[...], x_ref[...], (((1,), (1,)), ((), ())),
        preferred_element_type=jnp.float32)
    lg_ref[...] = logits


def _route_chunk(lg_v, oh_v, mp_v):
    @pl.loop(0, CHUNK // L)
    def _(g):
        sl = pl.ds(g * L, L)
        m = lg_v[0, sl]
        for e in range(1, NUM_EXPERTS):
            m = jnp.maximum(m, lg_v[e, sl])
        s = jnp.exp(lg_v[0, sl] - m)
        for e in range(1, NUM_EXPERTS):
            s = s + jnp.exp(lg_v[e, sl] - m)
        mp = jnp.exp(lg_v[0, sl] - m) / s
        idx = jnp.zeros((L,), jnp.int32)
        for e in range(1, NUM_EXPERTS):
            p = jnp.exp(lg_v[e, sl] - m) / s
            gt = p > mp
            mp = jnp.where(gt, p, mp)
            idx = jnp.where(gt, e, idx)
        mp_v[0, sl] = mp
        for e in range(NUM_EXPERTS):
            oh_v[e, sl] = jnp.where(idx == e, 1, 0).astype(jnp.int32)


def _sc_router(lg):
    mesh = plsc.VectorSubcoreMesh(core_axis_name="c", subcore_axis_name="s")

    @functools.partial(
        pl.kernel, mesh=mesh,
        out_type=[
            jax.ShapeDtypeStruct((NUM_EXPERTS, N_TOK), jnp.int32),
            jax.ShapeDtypeStruct((1, N_TOK), jnp.float32),
        ],
        scratch_types=[
            pltpu.VMEM((NUM_EXPERTS, CHUNK), jnp.float32),
            pltpu.VMEM((NUM_EXPERTS, CHUNK), jnp.float32),
            pltpu.VMEM((NUM_EXPERTS, CHUNK), jnp.int32),
            pltpu.VMEM((1, CHUNK), jnp.float32),
            pltpu.SemaphoreType.DMA,
            pltpu.SemaphoreType.DMA,
        ],
        compiler_params=pltpu.CompilerParams(use_tc_tiling_on_sc=True),
    )
    def k(lg_hbm, oh_hbm, mp_hbm, lg_a, lg_b, oh_v, mp_v, sem_in, sem_out):
        wid = lax.axis_index("s") * NC + lax.axis_index("c")
        base = wid * TPW
        bufs = (lg_a, lg_b)
        cp0 = pltpu.async_copy(lg_hbm.at[:, pl.ds(base, CHUNK)], lg_a, sem_in)
        for c in range(N_CHUNKS):
            cur = bufs[c % 2]
            if c == 0:
                cp0.wait()
            else:
                pltpu.async_copy(
                    lg_hbm.at[:, pl.ds(base + c * CHUNK, CHUNK)],
                    cur, sem_in).wait()
            if c + 1 < N_CHUNKS:
                nxt = bufs[(c + 1) % 2]
                pltpu.async_copy(
                    lg_hbm.at[:, pl.ds(base + (c + 1) * CHUNK, CHUNK)],
                    nxt, sem_in)
            _route_chunk(cur, oh_v, mp_v)
            pltpu.sync_copy(oh_v, oh_hbm.at[:, pl.ds(base + c * CHUNK, CHUNK)])
            pltpu.sync_copy(mp_v, mp_hbm.at[:, pl.ds(base + c * CHUNK, CHUNK)])

    return k(lg)


def kernel(x, W):
    n = x.shape[0]
    lg_t = pl.pallas_call(
        _matmul_body,
        grid=(n // BLOCK_T,),
        in_specs=[
            pl.BlockSpec((BLOCK_T, D_MODEL), lambda i: (i, 0)),
            pl.BlockSpec((NUM_EXPERTS, D_MODEL), lambda i: (0, 0)),
        ],
        out_specs=pl.BlockSpec((NUM_EXPERTS, BLOCK_T), lambda i: (0, i)),
        out_shape=jax.ShapeDtypeStruct((NUM_EXPERTS, n), jnp.float32),
    )(x, W)
    oh_t, mp_t = _sc_router(lg_t)
    return oh_t.T, mp_t.T, lg_t.T


# speedup vs baseline: 1.7491x; 1.7491x over previous
import jax
import jax.numpy as jnp
from jax.experimental import pallas as pl

NUM_EXPERTS = 64
D_MODEL = 2048
BLOCK_T = 1024


def _router_body(x_ref, w_ref, oh_ref, mp_ref, lg_ref):
    x = x_ref[...]
    w = w_ref[...]
    logits = jax.lax.dot_general(
        w, x, (((1,), (1,)), ((), ())),
        preferred_element_type=jnp.float32)
    m = jnp.max(logits, axis=0, keepdims=True)
    e = jnp.exp(logits - m)
    s = jnp.sum(e, axis=0, keepdims=True)
    probs = e / s
    mp = jnp.max(probs, axis=0, keepdims=True)
    row = jax.lax.broadcasted_iota(jnp.int32, probs.shape, 0)
    idx = jnp.min(jnp.where(probs == mp, row, NUM_EXPERTS),
                  axis=0, keepdims=True)
    oh_ref[...] = (row == idx).astype(jnp.int32)
    mp_ref[...] = mp
    lg_ref[...] = logits


def kernel(x, W):
    n = x.shape[0]
    oh_t, mp_t, lg_t = pl.pallas_call(
        _router_body,
        grid=(n // BLOCK_T,),
        in_specs=[
            pl.BlockSpec((BLOCK_T, D_MODEL), lambda i: (i, 0)),
            pl.BlockSpec((NUM_EXPERTS, D_MODEL), lambda i: (0, 0)),
        ],
        out_specs=[
            pl.BlockSpec((NUM_EXPERTS, BLOCK_T), lambda i: (0, i)),
            pl.BlockSpec((1, BLOCK_T), lambda i: (0, i)),
            pl.BlockSpec((NUM_EXPERTS, BLOCK_T), lambda i: (0, i)),
        ],
        out_shape=[
            jax.ShapeDtypeStruct((NUM_EXPERTS, n), jnp.int32),
            jax.ShapeDtypeStruct((1, n), jnp.float32),
            jax.ShapeDtypeStruct((NUM_EXPERTS, n), jnp.float32),
        ],
    )(x, W)
    return oh_t.T, mp_t.T, lg_t.T
